# baseline (speedup 1.0000x reference)
import jax
import jax.numpy as jnp
from jax.experimental import pallas as pl

N_NODES = 10000
N_RELS = 500
N_EDGES = 320000
IN_FEATS = 128
OUT_FEATS = 128
H = 8
DH = OUT_FEATS // H
ALPHA = 0.15
HOP_NUM = 3
NEG_SLOPE = 0.2


def _matmul_kernel(x_ref, wt_ref, o_ref):
    o_ref[...] = jnp.dot(x_ref[...], wt_ref[...],
                         preferred_element_type=jnp.float32)


def _pallas_matmul(x, w_t, block_rows=512):
    n = x.shape[0]
    pad = (-n) % block_rows
    xp = jnp.pad(x, ((0, pad), (0, 0))) if pad else x
    np_ = xp.shape[0]
    out = pl.pallas_call(
        _matmul_kernel,
        grid=(np_ // block_rows,),
        in_specs=[
            pl.BlockSpec((block_rows, x.shape[1]), lambda i: (i, 0)),
            pl.BlockSpec((w_t.shape[0], w_t.shape[1]), lambda i: (0, 0)),
        ],
        out_specs=pl.BlockSpec((block_rows, w_t.shape[1]), lambda i: (i, 0)),
        out_shape=jax.ShapeDtypeStruct((np_, w_t.shape[1]), jnp.float32),
    )(xp, w_t)
    return out[:n] if pad else out


def _leaky_relu(x, slope):
    return jnp.where(x >= 0, x, slope * x)


def kernel(ent_embed, rel_embed, edge_index, e_label, W_ent, W_rel, W_ent_out,
           attn_h, attn_t, attn_r, w1, b1, w2, b2, rw_ent_1, rw_ent_2, rw_rel):
    src = edge_index[0]
    dst = edge_index[1]
    N = ent_embed.shape[0]

    rel_feat = _pallas_matmul(rel_embed, W_rel.T).reshape(-1, H, DH)
    ent_feat = _pallas_matmul(ent_embed, W_ent.T).reshape(-1, H, DH)

    eh = jnp.sum(ent_feat * attn_h, axis=-1)
    et = jnp.sum(ent_feat * attn_t, axis=-1)
    er_rel = jnp.sum(rel_feat * attn_r, axis=-1)
    er = er_rel[e_label]

    e = _leaky_relu(eh[src] + et[dst] + er, NEG_SLOPE)
    e_max = jax.ops.segment_max(e, dst, num_segments=N)
    e_exp = jnp.exp(e - e_max[dst])
    e_sum = jax.ops.segment_sum(e_exp, dst, num_segments=N)
    a = e_exp / (e_sum[dst] + 1e-16)
    a = a[:, :, None]

    feat_0 = ent_feat
    feat = feat_0
    for _ in range(HOP_NUM):
        msg = feat[src] * a
        agg = jax.ops.segment_sum(msg, dst, num_segments=N)
        feat = (1.0 - ALPHA) * agg + ALPHA * feat_0

    ent_rst = feat.reshape(N, -1)
    rel_rst = rel_feat.reshape(rel_embed.shape[0], -1)
    ent_rst = _pallas_matmul(ent_rst, W_ent_out.T) * rw_ent_1[0]
    ent_rst = ent_embed + ent_rst
    rel_rst = rel_embed + rw_rel[0] * rel_rst
    ff = (_pallas_matmul(jax.nn.relu(_pallas_matmul(ent_rst, w1.T) + b1), w2.T)
          + b2) * rw_ent_2[0]
    ent_rst = ent_rst + ff
    return (ent_rst, rel_rst)
